# baseline (speedup 1.0000x reference)
import functools

import jax
import jax.numpy as jnp
from jax import lax
from jax.experimental import pallas as pl
from jax.experimental.pallas import tpu as pltpu
from jax.experimental.pallas import tpu_sc as plsc

_NUM_CHANNELS = 384
_NUM_BATCH = 16
_ROW = 56 * 56
_NROWS = _NUM_BATCH * _NUM_CHANNELS
_NC = 2
_NS = 16
_NW = _NC * _NS
_ROWS_PER_W = _NROWS // _NW
_CHUNK = 16
_NCHUNK = _ROWS_PER_W // _CHUNK
_LANES = 16


def _build_shuffle():
    mesh = plsc.VectorSubcoreMesh(core_axis_name="c", subcore_axis_name="s")

    @functools.partial(
        pl.kernel,
        mesh=mesh,
        out_type=jax.ShapeDtypeStruct((_NROWS, _ROW), jnp.float32),
        compiler_params=pltpu.CompilerParams(use_tc_tiling_on_sc=False),
        scratch_types=[
            pltpu.VMEM((_ROWS_PER_W,), jnp.int32),
            pltpu.VMEM((_CHUNK, _ROW), jnp.float32),
            pltpu.VMEM((_CHUNK, _ROW), jnp.float32),
            pltpu.SemaphoreType.DMA,
            pltpu.SemaphoreType.DMA,
            pltpu.SemaphoreType.DMA,
            pltpu.SemaphoreType.DMA,
        ],
    )
    def shuffle(x_hbm, idx_hbm, out_hbm, idx_v, buf0, buf1, g0, g1, s0, s1):
        wid = lax.axis_index("s") * _NC + lax.axis_index("c")
        batch = wid // 2
        c0 = (wid % 2) * _ROWS_PER_W
        row_base = wid * _ROWS_PER_W

        pltpu.sync_copy(idx_hbm.at[pl.ds(c0, _ROWS_PER_W)], idx_v)
        off = batch * _NUM_CHANNELS
        for j in range(_ROWS_PER_W // _LANES):
            sl = pl.ds(j * _LANES, _LANES)
            idx_v[sl] = idx_v[sl] + off

        bufs = (buf0, buf1)
        gsems = (g0, g1)
        ssems = (s0, s1)
        gh = [None] * _NCHUNK
        sh = [None] * _NCHUNK
        for k in range(2):
            gh[k] = pltpu.async_copy(
                x_hbm.at[idx_v.at[pl.ds(k * _CHUNK, _CHUNK)]],
                bufs[k], gsems[k])
        for k in range(_NCHUNK):
            p = k % 2
            gh[k].wait()
            sh[k] = pltpu.async_copy(
                bufs[p],
                out_hbm.at[pl.ds(row_base + k * _CHUNK, _CHUNK)],
                ssems[p])
            nxt = k + 2
            if nxt < _NCHUNK:
                sh[k].wait()
                gh[nxt] = pltpu.async_copy(
                    x_hbm.at[idx_v.at[pl.ds(nxt * _CHUNK, _CHUNK)]],
                    bufs[p], gsems[p])
        sh[_NCHUNK - 2].wait()
        sh[_NCHUNK - 1].wait()

    return shuffle


_shuffle = _build_shuffle()


def kernel(x, objective, indices, rev_indices):
    table = x.reshape(_NROWS, _ROW)
    out = _shuffle(table, indices)
    return (out.reshape(x.shape), objective)
